# Initial kernel scaffold; baseline (speedup 1.0000x reference)
import jax
import jax.numpy as jnp
from jax.experimental import pallas as pl


def kernel(input_seqs, pos_emb):
    raise NotImplementedError("write your pallas kernel here")


# speedup vs baseline: 26.5403x; 26.5403x over previous
import jax
import jax.numpy as jnp
from jax.experimental import pallas as pl


def _body(emb_ref, out_ref):
    out_ref[...] = jnp.broadcast_to(emb_ref[...], out_ref.shape)


def kernel(input_seqs, pos_emb):
    B, L = input_seqs.shape
    Lk, E = pos_emb.shape
    flat = pos_emb.reshape(1, Lk * E)
    BB = 256
    out = pl.pallas_call(
        _body,
        grid=(B // BB,),
        in_specs=[pl.BlockSpec((1, Lk * E), lambda i: (0, 0))],
        out_specs=pl.BlockSpec((BB, Lk * E), lambda i: (i, 0)),
        out_shape=jax.ShapeDtypeStruct((B, Lk * E), jnp.float32),
    )(flat)
    return out.reshape(B, L, E)
